# baseline (speedup 1.0000x reference)
import functools

import jax
import jax.numpy as jnp
from jax import lax
from jax.experimental import pallas as pl
from jax.experimental.pallas import tpu as pltpu
from jax.experimental.pallas import tpu_sc as plsc

M = 8192
T = 8
K = 2048
NW = 32
RW = M // NW
C = 4
NCHUNK = RW // C
GC = C * T


def _body(expert_hbm, w_hbm, inv_hbm, out_hbm, idx_v, w_v, rows_v, stage_v, sem):
    nc = 2
    wid = lax.axis_index("s") * nc + lax.axis_index("c")
    base_row = wid * RW
    base_g = base_row * T

    pltpu.sync_copy(inv_hbm.at[pl.ds(base_g, RW * T)], idx_v)

    def chunk_body(c, _):
        pltpu.sync_copy(w_hbm.at[pl.ds(base_g + c * GC, GC)], w_v)
        pltpu.async_copy(
            expert_hbm.at[idx_v.at[pl.ds(c * GC, GC)]], rows_v, sem
        ).wait()

        for r in range(C):
            row0 = r * T
            wsp = [w_v[row0 + t, :] for t in range(T)]

            def slice_body(s, _, row0=row0, wsp=wsp, r=r):
                off = s * 32
                acc_lo = None
                acc_hi = None
                for t in range(T):
                    v = plsc.bitcast(
                        rows_v[row0 + t, pl.ds(s * 16, 16)], jnp.bfloat16
                    )
                    lo, hi = plsc.unpack(
                        v,
                        format=plsc.PackFormat.INTERLEAVED,
                        preferred_element_type=jnp.float32,
                    )
                    if acc_lo is None:
                        acc_lo = lo * wsp[t]
                        acc_hi = hi * wsp[t]
                    else:
                        acc_lo = acc_lo + lo * wsp[t]
                        acc_hi = acc_hi + hi * wsp[t]
                out_bf = plsc.pack(
                    acc_lo,
                    acc_hi,
                    format=plsc.PackFormat.INTERLEAVED,
                    preferred_element_type=jnp.bfloat16,
                )
                stage_v[r, pl.ds(s * 16, 16)] = plsc.bitcast(out_bf, jnp.int32)
                return 0

            lax.fori_loop(0, K // 32, slice_body, 0)

        pltpu.sync_copy(stage_v, out_hbm.at[pl.ds(base_row + c * C, C)])
        return 0

    lax.fori_loop(0, NCHUNK, chunk_body, 0)


@jax.jit
def _run(expert_output, w_f32, inv_perm):
    mesh = plsc.VectorSubcoreMesh(core_axis_name="c", subcore_axis_name="s")
    return pl.kernel(
        _body,
        out_type=jax.ShapeDtypeStruct((M, K // 2), jnp.int32),
        mesh=mesh,
        compiler_params=pltpu.CompilerParams(needs_layout_passes=False),
        scratch_types=[
            pltpu.VMEM((RW * T,), jnp.int32),
            pltpu.VMEM((GC, 16), jnp.float32),
            pltpu.VMEM((GC, K // 2), jnp.int32),
            pltpu.VMEM((C, K // 2), jnp.int32),
            pltpu.SemaphoreType.DMA,
        ],
    )(expert_output, w_f32, inv_perm)


def kernel(expert_output, topk_vals, inv_perm):
    w_f32 = jnp.broadcast_to(
        topk_vals.astype(jnp.float32).reshape(M * T, 1), (M * T, 16)
    )
    expert_i32 = jax.lax.bitcast_convert_type(
        expert_output.reshape(M * T, K // 2, 2), jnp.int32
    )
    out_i32 = _run(expert_i32, w_f32, inv_perm)
    return jax.lax.bitcast_convert_type(out_i32, jnp.bfloat16).reshape(M, K)


# speedup vs baseline: 1.0908x; 1.0908x over previous
import jax
import jax.numpy as jnp
from jax import lax
from jax.experimental import pallas as pl
from jax.experimental.pallas import tpu as pltpu
from jax.experimental.pallas import tpu_sc as plsc

M = 8192
T = 8
K = 2048
KW = K // 2
NW = 32
RW = M // NW
C = 4
NCHUNK = RW // C
NPAIR = NCHUNK // 2
GC = C * T
NS = K // 32


def _body(expert_hbm, w_hbm, inv_hbm, out_hbm,
          idx_v, w_v, buf0, buf1, stage0, stage1,
          gsem0, gsem1, wsem0, wsem1):
    nc = 2
    wid = lax.axis_index("s") * nc + lax.axis_index("c")
    base_row = wid * RW
    base_g = base_row * T

    pltpu.sync_copy(inv_hbm.at[pl.ds(base_g, RW * T)], idx_v)

    def start_gather(c, buf, sem):
        return pltpu.async_copy(
            expert_hbm.at[idx_v.at[pl.ds(c * GC, GC)]], buf, sem
        )

    def compute(c, buf, stage):
        pltpu.sync_copy(w_hbm.at[pl.ds(base_g + c * GC, GC)], w_v)
        for r in range(C):
            row0 = r * T
            wsp = [
                plsc.bitcast(w_v[row0 + t, :], jnp.bfloat16)
                for t in range(T)
            ]

            @plsc.parallel_loop(0, NS, unroll=4)
            def slice_body(s, row0=row0, wsp=wsp, r=r):
                off = s * 16
                p = [
                    plsc.bitcast(buf[row0 + t, pl.ds(off, 16)], jnp.bfloat16)
                    * wsp[t]
                    for t in range(T)
                ]
                s0 = (p[0] + p[1]) + (p[2] + p[3])
                s1 = (p[4] + p[5]) + (p[6] + p[7])
                stage[r, pl.ds(off, 16)] = plsc.bitcast(s0 + s1, jnp.int32)

    start_gather(0, buf0, gsem0)

    def pair_body(cp, _):
        c0 = cp * 2
        c1 = c0 + 1
        g1 = start_gather(c1, buf1, gsem1)
        pltpu.make_async_copy(
            expert_hbm.at[idx_v.at[pl.ds(c0 * GC, GC)]], buf0, gsem0
        ).wait()
        compute(c0, buf0, stage0)
        pltpu.sync_copy(stage0, out_hbm.at[pl.ds(base_row + c0 * C, C)])

        start_gather(jnp.where(c0 + 2 < NCHUNK, c0 + 2, 0), buf0, gsem0)

        g1.wait()
        compute(c1, buf1, stage1)
        pltpu.sync_copy(stage1, out_hbm.at[pl.ds(base_row + c1 * C, C)])
        return 0

    lax.fori_loop(0, NPAIR, pair_body, 0)
    pltpu.make_async_copy(
        expert_hbm.at[idx_v.at[pl.ds(0, GC)]], buf0, gsem0
    ).wait()


@jax.jit
def _run(expert_i32, w_pack, inv_perm):
    mesh = plsc.VectorSubcoreMesh(core_axis_name="c", subcore_axis_name="s")
    return pl.kernel(
        _body,
        out_type=jax.ShapeDtypeStruct((M, KW), jnp.int32),
        mesh=mesh,
        compiler_params=pltpu.CompilerParams(needs_layout_passes=False),
        scratch_types=[
            pltpu.VMEM((RW * T,), jnp.int32),
            pltpu.VMEM((GC, 16), jnp.int32),
            pltpu.VMEM((GC, KW), jnp.int32),
            pltpu.VMEM((GC, KW), jnp.int32),
            pltpu.VMEM((C, KW), jnp.int32),
            pltpu.VMEM((C, KW), jnp.int32),
            pltpu.SemaphoreType.DMA,
            pltpu.SemaphoreType.DMA,
            pltpu.SemaphoreType.DMA,
            pltpu.SemaphoreType.DMA,
        ],
    )(expert_i32, w_pack, inv_perm)


def kernel(expert_output, topk_vals, inv_perm):
    w16 = jax.lax.bitcast_convert_type(topk_vals, jnp.uint16).astype(jnp.uint32)
    w32 = ((w16 << 16) | w16).astype(jnp.int32).reshape(M * T, 1)
    w_pack = jnp.broadcast_to(w32, (M * T, 16))
    expert_i32 = jax.lax.bitcast_convert_type(
        expert_output.reshape(M * T, KW, 2), jnp.int32
    )
    out_i32 = _run(expert_i32, w_pack, inv_perm)
    return jax.lax.bitcast_convert_type(out_i32, jnp.bfloat16).reshape(M, K)
